# baseline (speedup 1.0000x reference)
import functools

import jax
import jax.numpy as jnp
from jax import lax
from jax.experimental import pallas as pl
from jax.experimental.pallas import tpu as pltpu
from jax.experimental.pallas import tpu_sc as plsc

N = 10000
E = 320000
NG = 200
EG = 600
G = 30
B = 64
H = 128
OUT = 16

NS = 16
SLICE = 640
NPAD = NS * SLICE
CH = 128
TPE = E // NS
NCHUNK = TPE // CH
TAIL = TPE - NCHUNK * CH
NGPAD = 256
GCH = NGPAD // CH
GEC = 5
GPAD = 128


def _rsqrt16(x):
    i = lax.bitcast_convert_type(x, jnp.int32)
    i = jnp.int32(0x5F3759DF) - lax.shift_right_logical(i, 1)
    y = lax.bitcast_convert_type(i, jnp.float32)
    for _ in range(3):
        y = y * (1.5 - 0.5 * x * y * y)
    return y


def _sc_body(src_h, dst_h, gsrc_h, gdst_h, gseg_h, fetch_h,
             t2f_h, m_h,
             sidx_v, didx_v, xtab_v, vals_v, slice_v, invs_v, work_v,
             zer_v, ones_v,
             gsidx_v, gdidx_v, gseg_v, gx_v, ginvs_v, gt_v, gread_v,
             gvals_v, fidx_v, fout_v, mout_v, cnt_v, valsbuf_v, dumi_v,
             acc_sh, x_sh, gacc_sh, scnt_sh, ssum_sh, sem, sem2):
    tid = lax.axis_index("s")
    base = tid * SLICE
    ebase = tid * TPE

    def _drain_one(j, _):
        pltpu.make_async_copy(src_h.at[pl.ds(0, CH)], dumi_v, sem).wait()
        return 0

    def _drain_tail():
        pltpu.make_async_copy(src_h.at[pl.ds(0, TAIL)],
                              dumi_v.at[pl.ds(0, TAIL)], sem).wait()

    for i in range(SLICE // 16):
        zer_v[pl.ds(16 * i, 16)] = jnp.zeros((16,), jnp.float32)
    for i in range(CH // 16):
        ones_v[pl.ds(16 * i, 16)] = jnp.ones((16,), jnp.float32)

    is0 = tid == 0

    src_cp = pltpu.async_copy(src_h.at[pl.ds(ebase, TPE)], sidx_v, sem2)
    pltpu.sync_copy(dst_h.at[pl.ds(ebase, TPE)], didx_v)

    pltpu.sync_copy(zer_v, acc_sh.at[pl.ds(base, SLICE)])

    @pl.when(is0)
    def _g_stage():
        pltpu.sync_copy(gsrc_h, gsidx_v)
        pltpu.sync_copy(gdst_h, gdidx_v)
        pltpu.sync_copy(gseg_h, gseg_v)
        pltpu.sync_copy(fetch_h, fidx_v)
        pltpu.sync_copy(zer_v.at[pl.ds(0, NGPAD)], gacc_sh)
    plsc.subcore_barrier()

    def _g_read_transform(which):
        pltpu.sync_copy(gacc_sh, gread_v)
        for i in range(NGPAD // 16):
            u = gread_v[pl.ds(16 * i, 16)]
            if which == 0:
                y = _rsqrt16(jnp.maximum(u, 1.0))
                ginvs_v[pl.ds(16 * i, 16)] = y
                gx_v[pl.ds(16 * i, 16)] = u * y
            elif which == 1:
                y = ginvs_v[pl.ds(16 * i, 16)]
                gx_v[pl.ds(16 * i, 16)] = u * y * y
            else:
                y = ginvs_v[pl.ds(16 * i, 16)]
                gt_v[pl.ds(16 * i, 16)] = u * y
        if which != 2:
            pltpu.sync_copy(zer_v.at[pl.ds(0, NGPAD)], gacc_sh)

    def _g_drain():
        for _ in range(GEC):
            _drain_one(0, 0)

    def _passA(j, _):
        pltpu.async_copy(ones_v,
                         acc_sh.at[didx_v.at[pl.ds(j * CH, CH)]],
                         sem, add=True)
        return 0
    lax.fori_loop(0, NCHUNK, _passA, 0)
    pltpu.async_copy(ones_v.at[pl.ds(0, TAIL)],
                     acc_sh.at[didx_v.at[pl.ds(NCHUNK * CH, TAIL)]],
                     sem, add=True)

    @pl.when(is0)
    def _gA():
        for j in range(GEC):
            pltpu.async_copy(ones_v, gacc_sh.at[gdidx_v.at[j]], sem, add=True)

    lax.fori_loop(0, NCHUNK, _drain_one, 0)
    _drain_tail()

    @pl.when(is0)
    def _gA2():
        _g_drain()
        _g_read_transform(0)

    src_cp.wait()
    plsc.subcore_barrier()

    pltpu.sync_copy(acc_sh.at[pl.ds(base, SLICE)], slice_v)
    for i in range(SLICE // 16):
        d = slice_v[pl.ds(16 * i, 16)]
        y = _rsqrt16(jnp.maximum(d, 1.0))
        invs_v[pl.ds(16 * i, 16)] = y
        work_v[pl.ds(16 * i, 16)] = d * y
    pltpu.sync_copy(zer_v, acc_sh.at[pl.ds(base, SLICE)])
    pltpu.sync_copy(work_v, x_sh.at[pl.ds(base, SLICE)])
    plsc.subcore_barrier()
    pltpu.sync_copy(x_sh, xtab_v)

    def _gather_scatter(j, _):
        off = j * CH
        for i in range(CH // 16):
            idx = sidx_v[pl.ds(off + 16 * i, 16)]
            valsbuf_v[pl.ds(off + 16 * i, 16)] = plsc.load_gather(xtab_v, [idx])
        pltpu.async_copy(valsbuf_v.at[pl.ds(off, CH)],
                         acc_sh.at[didx_v.at[pl.ds(off, CH)]],
                         sem, add=True)
        return 0

    def _gather_scatter_tail():
        off = NCHUNK * CH
        for i in range(TAIL // 16):
            idx = sidx_v[pl.ds(off + 16 * i, 16)]
            valsbuf_v[pl.ds(off + 16 * i, 16)] = plsc.load_gather(xtab_v, [idx])
        pltpu.async_copy(valsbuf_v.at[pl.ds(off, TAIL)],
                         acc_sh.at[didx_v.at[pl.ds(off, TAIL)]],
                         sem, add=True)

    def _g_gather_scatter():
        for j in range(GEC):
            grow = gsidx_v.at[j]
            gvrow = gvals_v.at[j]
            for i in range(CH // 16):
                idx = grow[pl.ds(16 * i, 16)]
                gvrow[pl.ds(16 * i, 16)] = plsc.load_gather(gx_v, [idx])
            pltpu.async_copy(gvrow, gacc_sh.at[gdidx_v.at[j]], sem, add=True)

    lax.fori_loop(0, NCHUNK, _gather_scatter, 0)
    _gather_scatter_tail()

    @pl.when(is0)
    def _gB():
        _g_gather_scatter()

    lax.fori_loop(0, NCHUNK, _drain_one, 0)
    _drain_tail()

    @pl.when(is0)
    def _gB2():
        _g_drain()
        _g_read_transform(1)

    plsc.subcore_barrier()

    pltpu.sync_copy(acc_sh.at[pl.ds(base, SLICE)], slice_v)
    for i in range(SLICE // 16):
        u = slice_v[pl.ds(16 * i, 16)]
        y = invs_v[pl.ds(16 * i, 16)]
        work_v[pl.ds(16 * i, 16)] = u * y * y
    pltpu.sync_copy(zer_v, acc_sh.at[pl.ds(base, SLICE)])
    pltpu.sync_copy(work_v, x_sh.at[pl.ds(base, SLICE)])
    plsc.subcore_barrier()
    pltpu.sync_copy(x_sh, xtab_v)

    lax.fori_loop(0, NCHUNK, _gather_scatter, 0)
    _gather_scatter_tail()

    @pl.when(is0)
    def _gC():
        _g_gather_scatter()

    lax.fori_loop(0, NCHUNK, _drain_one, 0)
    _drain_tail()

    @pl.when(is0)
    def _gC2():
        _g_drain()
        _g_read_transform(2)

    plsc.subcore_barrier()

    pltpu.sync_copy(acc_sh.at[pl.ds(base, SLICE)], slice_v)
    for i in range(SLICE // 16):
        u = slice_v[pl.ds(16 * i, 16)]
        y = invs_v[pl.ds(16 * i, 16)]
        work_v[pl.ds(16 * i, 16)] = u * y
    pltpu.sync_copy(work_v, x_sh.at[pl.ds(base, SLICE)])
    plsc.subcore_barrier()

    @pl.when(tid == 0)
    def _tail():
        pltpu.sync_copy(x_sh, xtab_v)
        pltpu.sync_copy(fetch_h, fidx_v)
        for i in range(B // 16):
            idx = fidx_v[pl.ds(16 * i, 16)]
            fout_v[pl.ds(16 * i, 16)] = plsc.load_gather(xtab_v, [idx])
        pltpu.sync_copy(fout_v, t2f_h)

        pltpu.sync_copy(gsrc_h, gsidx_v)
        pltpu.sync_copy(gdst_h, gdidx_v)
        pltpu.sync_copy(gseg_h, gseg_v)

        pltpu.sync_copy(zer_v.at[pl.ds(0, NGPAD)], gacc_sh)
        for j in range(GEC):
            pltpu.sync_copy(ones_v, gacc_sh.at[gdidx_v.at[j]], add=True)
        pltpu.sync_copy(gacc_sh, gread_v)
        for i in range(NGPAD // 16):
            d = gread_v[pl.ds(16 * i, 16)]
            y = _rsqrt16(jnp.maximum(d, 1.0))
            ginvs_v[pl.ds(16 * i, 16)] = y
            gx_v[pl.ds(16 * i, 16)] = d * y

        pltpu.sync_copy(zer_v.at[pl.ds(0, NGPAD)], gacc_sh)
        for j in range(GEC):
            grow = gsidx_v.at[j]
            for i in range(CH // 16):
                idx = grow[pl.ds(16 * i, 16)]
                vals_v[pl.ds(16 * i, 16)] = plsc.load_gather(gx_v, [idx])
            pltpu.sync_copy(vals_v, gacc_sh.at[gdidx_v.at[j]], add=True)
        pltpu.sync_copy(gacc_sh, gread_v)
        for i in range(NGPAD // 16):
            u = gread_v[pl.ds(16 * i, 16)]
            y = ginvs_v[pl.ds(16 * i, 16)]
            gx_v[pl.ds(16 * i, 16)] = u * y * y

        pltpu.sync_copy(zer_v.at[pl.ds(0, NGPAD)], gacc_sh)
        for j in range(GEC):
            grow = gsidx_v.at[j]
            for i in range(CH // 16):
                idx = grow[pl.ds(16 * i, 16)]
                vals_v[pl.ds(16 * i, 16)] = plsc.load_gather(gx_v, [idx])
            pltpu.sync_copy(vals_v, gacc_sh.at[gdidx_v.at[j]], add=True)
        pltpu.sync_copy(gacc_sh, gread_v)
        for i in range(NGPAD // 16):
            u = gread_v[pl.ds(16 * i, 16)]
            y = ginvs_v[pl.ds(16 * i, 16)]
            gt_v[pl.ds(16 * i, 16)] = u * y

        pltpu.sync_copy(zer_v.at[pl.ds(0, GPAD)], scnt_sh)
        pltpu.sync_copy(zer_v.at[pl.ds(0, GPAD)], ssum_sh)
        for j in range(GCH):
            pltpu.sync_copy(ones_v, scnt_sh.at[gseg_v.at[j]], add=True)
            pltpu.sync_copy(gt_v.at[pl.ds(j * CH, CH)],
                            ssum_sh.at[gseg_v.at[j]], add=True)
        pltpu.sync_copy(scnt_sh, cnt_v)
        pltpu.sync_copy(ssum_sh, gread_v.at[pl.ds(0, GPAD)])
        for i in range(GPAD // 16):
            c = cnt_v[pl.ds(16 * i, 16)]
            s = gread_v[pl.ds(16 * i, 16)]
            mout_v[pl.ds(16 * i, 16)] = s / jnp.maximum(c, 1.0)
        pltpu.sync_copy(mout_v, m_h)


def _sc_call(*args):
    return functools.partial(
        pl.kernel,
        out_type=(jax.ShapeDtypeStruct((B,), jnp.float32),
                  jax.ShapeDtypeStruct((GPAD,), jnp.float32)),
        mesh=plsc.VectorSubcoreMesh(core_axis_name="c", subcore_axis_name="s",
                                    num_cores=1, num_subcores=NS),
        compiler_params=pltpu.CompilerParams(needs_layout_passes=False),
        scratch_types=[
        pltpu.VMEM((TPE,), jnp.int32),
        pltpu.VMEM((TPE,), jnp.int32),
        pltpu.VMEM((NPAD,), jnp.float32),
        pltpu.VMEM((CH,), jnp.float32),
        pltpu.VMEM((SLICE,), jnp.float32),
        pltpu.VMEM((SLICE,), jnp.float32),
        pltpu.VMEM((SLICE,), jnp.float32),
        pltpu.VMEM((SLICE,), jnp.float32),
        pltpu.VMEM((CH,), jnp.float32),
        pltpu.VMEM((GEC, CH), jnp.int32),
        pltpu.VMEM((GEC, CH), jnp.int32),
        pltpu.VMEM((GCH, CH), jnp.int32),
        pltpu.VMEM((NGPAD,), jnp.float32),
        pltpu.VMEM((NGPAD,), jnp.float32),
        pltpu.VMEM((NGPAD,), jnp.float32),
        pltpu.VMEM((NGPAD,), jnp.float32),
        pltpu.VMEM((GEC, CH), jnp.float32),
        pltpu.VMEM((B,), jnp.int32),
        pltpu.VMEM((B,), jnp.float32),
        pltpu.VMEM((GPAD,), jnp.float32),
        pltpu.VMEM((GPAD,), jnp.float32),
        pltpu.VMEM((TPE,), jnp.float32),
        pltpu.VMEM((CH,), jnp.int32),
        pltpu.VMEM_SHARED((NPAD,), jnp.float32),
        pltpu.VMEM_SHARED((NPAD,), jnp.float32),
        pltpu.VMEM_SHARED((NGPAD,), jnp.float32),
        pltpu.VMEM_SHARED((GPAD,), jnp.float32),
        pltpu.VMEM_SHARED((GPAD,), jnp.float32),
        pltpu.SemaphoreType.DMA,
        pltpu.SemaphoreType.DMA,
        ],
    )(_sc_body)(*args)


def _dotT(a, b):
    return lax.dot_general(a, b, (((1,), (1,)), ((), ())),
                           preferred_element_type=jnp.float32,
                           precision=lax.Precision.HIGHEST)


def _tc_body(t2f_ref, m_ref, w1_ref, w2_ref, wq_ref, bq_ref, wk_ref,
             bk_ref, wv_ref, bv_ref, wl_ref, bl_ref,
             wlin_ref, blin_ref, out_ref):
    w1p = jnp.maximum(w1_ref[...], 0.0)
    rv = jnp.dot(w1p, w2_ref[...], preferred_element_type=jnp.float32,
                 precision=lax.Precision.HIGHEST)
    rv = jnp.maximum(rv, 0.0)
    hf = t2f_ref[...] * rv
    hg = m_ref[0:32, :] * rv
    q = _dotT(hf, wq_ref[...]) + bq_ref[...]
    k = _dotT(hg, wk_ref[...]) + bk_ref[...]
    v = _dotT(hg, wv_ref[...]) + bv_ref[...]
    s = _dotT(q, k)
    col = lax.broadcasted_iota(jnp.int32, (B, 32), 1)
    s = jnp.where(col < G, s, -1e30)
    s = s - jnp.max(s, axis=1, keepdims=True)
    p = jnp.exp(s)
    p = p / jnp.sum(p, axis=1, keepdims=True)
    ctx = jnp.dot(p, v, preferred_element_type=jnp.float32,
                  precision=lax.Precision.HIGHEST)
    wl = wl_ref[...]
    ho = _dotT(ctx, wl[:, :H]) + _dotT(hf, wl[:, H:]) + bl_ref[...]
    out_ref[...] = _dotT(ho, wlin_ref[...]) + blin_ref[...]


def kernel(edge_index, fetch_idx, g_edge_index, g_seg_ids, w1, b1, w2, b2,
           w_q, b_q, w_k, b_k, w_v, b_v, w_l, b_l, w_lin, b_lin):
    f32 = jnp.float32
    i32 = jnp.int32
    srcp = edge_index[0].astype(i32)
    dstp = edge_index[1].astype(i32)
    gpad = GEC * CH - EG
    gsrcp = jnp.concatenate([g_edge_index[0].astype(i32),
                             jnp.zeros((gpad,), i32)]).reshape(GEC, CH)
    gdstp = jnp.concatenate([g_edge_index[1].astype(i32),
                             jnp.full((gpad,), NGPAD - 1, i32)]).reshape(GEC, CH)
    spad = GCH * CH - NG
    gsegp = jnp.concatenate([g_seg_ids.astype(i32),
                             jnp.full((spad,), GPAD - 1, i32)]).reshape(GCH, CH)

    t2f, m = _sc_call(srcp, dstp, gsrcp, gdstp, gsegp, fetch_idx.astype(i32))

    return pl.pallas_call(
        _tc_body,
        out_shape=jax.ShapeDtypeStruct((B, OUT), f32),
    )(t2f.reshape(B, 1), m.reshape(GPAD, 1), w1, w2,
      w_q, b_q.reshape(1, H), w_k, b_k.reshape(1, H),
      w_v, b_v.reshape(1, H),
      w_l, b_l.reshape(1, H),
      w_lin, b_lin.reshape(1, OUT))


# speedup vs baseline: 1.0750x; 1.0750x over previous
import functools

import jax
import jax.numpy as jnp
from jax import lax
from jax.experimental import pallas as pl
from jax.experimental.pallas import tpu as pltpu
from jax.experimental.pallas import tpu_sc as plsc

N = 10000
E = 320000
NG = 200
EG = 600
G = 30
B = 64
H = 128
OUT = 16

NS = 16
SLICE = 640
NPAD = NS * SLICE
CH = 128
TPE = E // NS
NCHUNK = TPE // CH
TAIL = TPE - NCHUNK * CH
NGPAD = 256
GCH = NGPAD // CH
GEC = 5
GPAD = 128


def _rsqrt16(x):
    i = lax.bitcast_convert_type(x, jnp.int32)
    i = jnp.int32(0x5F3759DF) - lax.shift_right_logical(i, 1)
    y = lax.bitcast_convert_type(i, jnp.float32)
    for _ in range(3):
        y = y * (1.5 - 0.5 * x * y * y)
    return y


def _sc_body(src_h, dst_h, gsrc_h, gdst_h, gseg_h, fetch_h,
             t2f_h, m_h,
             sidx_v, didx_v, xtab_v, vals_v, slice_v, invs_v, work_v,
             zer_v, ones_v,
             gsidx_v, gdidx_v, gseg_v, gx_v, ginvs_v, gt_v, gread_v,
             gvals_v, fidx_v, fout_v, mout_v, cnt_v, valsbuf_v, dumi_v,
             acc_sh, x_sh, gacc_sh, scnt_sh, ssum_sh, sem, sem2):
    tid = lax.axis_index("s")
    base = tid * SLICE
    ebase = tid * TPE

    def _drain_one(j, _):
        pltpu.make_async_copy(src_h.at[pl.ds(0, CH)], dumi_v, sem).wait()
        return 0

    def _drain_tail():
        pltpu.make_async_copy(src_h.at[pl.ds(0, TAIL)],
                              dumi_v.at[pl.ds(0, TAIL)], sem).wait()

    for i in range(SLICE // 16):
        zer_v[pl.ds(16 * i, 16)] = jnp.zeros((16,), jnp.float32)
    for i in range(CH // 16):
        ones_v[pl.ds(16 * i, 16)] = jnp.ones((16,), jnp.float32)

    is0 = tid == 0

    src_cp = pltpu.async_copy(src_h.at[pl.ds(ebase, TPE)], sidx_v, sem2)
    pltpu.sync_copy(dst_h.at[pl.ds(ebase, TPE)], didx_v)

    pltpu.sync_copy(zer_v, acc_sh.at[pl.ds(base, SLICE)])

    @pl.when(is0)
    def _g_stage():
        pltpu.sync_copy(gsrc_h, gsidx_v)
        pltpu.sync_copy(gdst_h, gdidx_v)
        pltpu.sync_copy(gseg_h, gseg_v)
        pltpu.sync_copy(fetch_h, fidx_v)
        pltpu.sync_copy(zer_v.at[pl.ds(0, NGPAD)], gacc_sh)
    plsc.subcore_barrier()

    def _g_read_transform(which):
        pltpu.sync_copy(gacc_sh, gread_v)
        for i in range(NGPAD // 16):
            u = gread_v[pl.ds(16 * i, 16)]
            if which == 0:
                y = _rsqrt16(jnp.maximum(u, 1.0))
                ginvs_v[pl.ds(16 * i, 16)] = y
                gx_v[pl.ds(16 * i, 16)] = u * y
            elif which == 1:
                y = ginvs_v[pl.ds(16 * i, 16)]
                gx_v[pl.ds(16 * i, 16)] = u * y * y
            else:
                y = ginvs_v[pl.ds(16 * i, 16)]
                gt_v[pl.ds(16 * i, 16)] = u * y
        if which != 2:
            pltpu.sync_copy(zer_v.at[pl.ds(0, NGPAD)], gacc_sh)

    def _g_drain():
        for _ in range(GEC):
            _drain_one(0, 0)

    def _passA(j, _):
        pltpu.async_copy(ones_v,
                         acc_sh.at[didx_v.at[pl.ds(j * CH, CH)]],
                         sem, add=True)
        return 0
    lax.fori_loop(0, NCHUNK, _passA, 0)
    pltpu.async_copy(ones_v.at[pl.ds(0, TAIL)],
                     acc_sh.at[didx_v.at[pl.ds(NCHUNK * CH, TAIL)]],
                     sem, add=True)

    @pl.when(is0)
    def _gA():
        for j in range(GEC):
            pltpu.async_copy(ones_v, gacc_sh.at[gdidx_v.at[j]], sem, add=True)

    lax.fori_loop(0, NCHUNK, _drain_one, 0)
    _drain_tail()

    @pl.when(is0)
    def _gA2():
        _g_drain()
        _g_read_transform(0)

    src_cp.wait()
    plsc.subcore_barrier()

    pltpu.sync_copy(acc_sh.at[pl.ds(base, SLICE)], slice_v)
    for i in range(SLICE // 16):
        d = slice_v[pl.ds(16 * i, 16)]
        y = _rsqrt16(jnp.maximum(d, 1.0))
        invs_v[pl.ds(16 * i, 16)] = y
        work_v[pl.ds(16 * i, 16)] = d * y
    pltpu.sync_copy(zer_v, acc_sh.at[pl.ds(base, SLICE)])
    pltpu.sync_copy(work_v, x_sh.at[pl.ds(base, SLICE)])
    plsc.subcore_barrier()
    pltpu.sync_copy(x_sh, xtab_v)

    def _gather_scatter(j, _):
        off = j * CH
        for i in range(CH // 16):
            idx = sidx_v[pl.ds(off + 16 * i, 16)]
            valsbuf_v[pl.ds(off + 16 * i, 16)] = plsc.load_gather(xtab_v, [idx])
        pltpu.async_copy(valsbuf_v.at[pl.ds(off, CH)],
                         acc_sh.at[didx_v.at[pl.ds(off, CH)]],
                         sem, add=True)
        return 0

    def _gather_scatter_tail():
        off = NCHUNK * CH
        for i in range(TAIL // 16):
            idx = sidx_v[pl.ds(off + 16 * i, 16)]
            valsbuf_v[pl.ds(off + 16 * i, 16)] = plsc.load_gather(xtab_v, [idx])
        pltpu.async_copy(valsbuf_v.at[pl.ds(off, TAIL)],
                         acc_sh.at[didx_v.at[pl.ds(off, TAIL)]],
                         sem, add=True)

    def _g_gather_scatter():
        for j in range(GEC):
            grow = gsidx_v.at[j]
            gvrow = gvals_v.at[j]
            for i in range(CH // 16):
                idx = grow[pl.ds(16 * i, 16)]
                gvrow[pl.ds(16 * i, 16)] = plsc.load_gather(gx_v, [idx])
            pltpu.async_copy(gvrow, gacc_sh.at[gdidx_v.at[j]], sem, add=True)

    lax.fori_loop(0, NCHUNK, _gather_scatter, 0)
    _gather_scatter_tail()

    @pl.when(is0)
    def _gB():
        _g_gather_scatter()

    lax.fori_loop(0, NCHUNK, _drain_one, 0)
    _drain_tail()

    @pl.when(is0)
    def _gB2():
        _g_drain()
        _g_read_transform(1)

    plsc.subcore_barrier()

    pltpu.sync_copy(acc_sh.at[pl.ds(base, SLICE)], slice_v)
    for i in range(SLICE // 16):
        u = slice_v[pl.ds(16 * i, 16)]
        y = invs_v[pl.ds(16 * i, 16)]
        work_v[pl.ds(16 * i, 16)] = u * y * y
    pltpu.sync_copy(zer_v, acc_sh.at[pl.ds(base, SLICE)])
    pltpu.sync_copy(work_v, x_sh.at[pl.ds(base, SLICE)])
    plsc.subcore_barrier()
    pltpu.sync_copy(x_sh, xtab_v)

    lax.fori_loop(0, NCHUNK, _gather_scatter, 0)
    _gather_scatter_tail()

    @pl.when(is0)
    def _gC():
        _g_gather_scatter()

    lax.fori_loop(0, NCHUNK, _drain_one, 0)
    _drain_tail()

    @pl.when(is0)
    def _gC2():
        _g_drain()
        _g_read_transform(2)

    plsc.subcore_barrier()

    pltpu.sync_copy(acc_sh.at[pl.ds(base, SLICE)], slice_v)
    for i in range(SLICE // 16):
        u = slice_v[pl.ds(16 * i, 16)]
        y = invs_v[pl.ds(16 * i, 16)]
        work_v[pl.ds(16 * i, 16)] = u * y
    pltpu.sync_copy(work_v, x_sh.at[pl.ds(base, SLICE)])
    plsc.subcore_barrier()

    @pl.when(tid == 0)
    def _tail():
        pltpu.sync_copy(x_sh, xtab_v)
        for i in range(B // 16):
            idx = fidx_v[pl.ds(16 * i, 16)]
            fout_v[pl.ds(16 * i, 16)] = plsc.load_gather(xtab_v, [idx])
        pltpu.sync_copy(fout_v, t2f_h)

        pltpu.sync_copy(zer_v.at[pl.ds(0, GPAD)], scnt_sh)
        pltpu.sync_copy(zer_v.at[pl.ds(0, GPAD)], ssum_sh)
        for j in range(GCH):
            pltpu.sync_copy(ones_v, scnt_sh.at[gseg_v.at[j]], add=True)
            pltpu.sync_copy(gt_v.at[pl.ds(j * CH, CH)],
                            ssum_sh.at[gseg_v.at[j]], add=True)
        pltpu.sync_copy(scnt_sh, cnt_v)
        pltpu.sync_copy(ssum_sh, gread_v.at[pl.ds(0, GPAD)])
        for i in range(GPAD // 16):
            c = cnt_v[pl.ds(16 * i, 16)]
            s = gread_v[pl.ds(16 * i, 16)]
            mout_v[pl.ds(16 * i, 16)] = s / jnp.maximum(c, 1.0)
        pltpu.sync_copy(mout_v, m_h)


def _sc_call(*args):
    return functools.partial(
        pl.kernel,
        out_type=(jax.ShapeDtypeStruct((B,), jnp.float32),
                  jax.ShapeDtypeStruct((GPAD,), jnp.float32)),
        mesh=plsc.VectorSubcoreMesh(core_axis_name="c", subcore_axis_name="s",
                                    num_cores=1, num_subcores=NS),
        compiler_params=pltpu.CompilerParams(needs_layout_passes=False),
        scratch_types=[
        pltpu.VMEM((TPE,), jnp.int32),
        pltpu.VMEM((TPE,), jnp.int32),
        pltpu.VMEM((NPAD,), jnp.float32),
        pltpu.VMEM((CH,), jnp.float32),
        pltpu.VMEM((SLICE,), jnp.float32),
        pltpu.VMEM((SLICE,), jnp.float32),
        pltpu.VMEM((SLICE,), jnp.float32),
        pltpu.VMEM((SLICE,), jnp.float32),
        pltpu.VMEM((CH,), jnp.float32),
        pltpu.VMEM((GEC, CH), jnp.int32),
        pltpu.VMEM((GEC, CH), jnp.int32),
        pltpu.VMEM((GCH, CH), jnp.int32),
        pltpu.VMEM((NGPAD,), jnp.float32),
        pltpu.VMEM((NGPAD,), jnp.float32),
        pltpu.VMEM((NGPAD,), jnp.float32),
        pltpu.VMEM((NGPAD,), jnp.float32),
        pltpu.VMEM((GEC, CH), jnp.float32),
        pltpu.VMEM((B,), jnp.int32),
        pltpu.VMEM((B,), jnp.float32),
        pltpu.VMEM((GPAD,), jnp.float32),
        pltpu.VMEM((GPAD,), jnp.float32),
        pltpu.VMEM((TPE,), jnp.float32),
        pltpu.VMEM((CH,), jnp.int32),
        pltpu.VMEM_SHARED((NPAD,), jnp.float32),
        pltpu.VMEM_SHARED((NPAD,), jnp.float32),
        pltpu.VMEM_SHARED((NGPAD,), jnp.float32),
        pltpu.VMEM_SHARED((GPAD,), jnp.float32),
        pltpu.VMEM_SHARED((GPAD,), jnp.float32),
        pltpu.SemaphoreType.DMA,
        pltpu.SemaphoreType.DMA,
        ],
    )(_sc_body)(*args)


def _dotT(a, b):
    return lax.dot_general(a, b, (((1,), (1,)), ((), ())),
                           preferred_element_type=jnp.float32,
                           precision=lax.Precision.HIGHEST)


def _tc_body(t2f_ref, m_ref, w1_ref, w2_ref, wq_ref, bq_ref, wk_ref,
             bk_ref, wv_ref, bv_ref, wl_ref, bl_ref,
             wlin_ref, blin_ref, out_ref):
    w1p = jnp.maximum(w1_ref[...], 0.0)
    rv = jnp.dot(w1p, w2_ref[...], preferred_element_type=jnp.float32,
                 precision=lax.Precision.HIGHEST)
    rv = jnp.maximum(rv, 0.0)
    hf = t2f_ref[...] * rv
    hg = m_ref[0:32, :] * rv
    q = _dotT(hf, wq_ref[...]) + bq_ref[...]
    k = _dotT(hg, wk_ref[...]) + bk_ref[...]
    v = _dotT(hg, wv_ref[...]) + bv_ref[...]
    s = _dotT(q, k)
    col = lax.broadcasted_iota(jnp.int32, (B, 32), 1)
    s = jnp.where(col < G, s, -1e30)
    s = s - jnp.max(s, axis=1, keepdims=True)
    p = jnp.exp(s)
    p = p / jnp.sum(p, axis=1, keepdims=True)
    ctx = jnp.dot(p, v, preferred_element_type=jnp.float32,
                  precision=lax.Precision.HIGHEST)
    wl = wl_ref[...]
    ho = _dotT(ctx, wl[:, :H]) + _dotT(hf, wl[:, H:]) + bl_ref[...]
    out_ref[...] = _dotT(ho, wlin_ref[...]) + blin_ref[...]


def kernel(edge_index, fetch_idx, g_edge_index, g_seg_ids, w1, b1, w2, b2,
           w_q, b_q, w_k, b_k, w_v, b_v, w_l, b_l, w_lin, b_lin):
    f32 = jnp.float32
    i32 = jnp.int32
    srcp = edge_index[0].astype(i32)
    dstp = edge_index[1].astype(i32)
    gpad = GEC * CH - EG
    gsrcp = jnp.concatenate([g_edge_index[0].astype(i32),
                             jnp.zeros((gpad,), i32)]).reshape(GEC, CH)
    gdstp = jnp.concatenate([g_edge_index[1].astype(i32),
                             jnp.full((gpad,), NGPAD - 1, i32)]).reshape(GEC, CH)
    spad = GCH * CH - NG
    gsegp = jnp.concatenate([g_seg_ids.astype(i32),
                             jnp.full((spad,), GPAD - 1, i32)]).reshape(GCH, CH)

    t2f, m = _sc_call(srcp, dstp, gsrcp, gdstp, gsegp, fetch_idx.astype(i32))

    return pl.pallas_call(
        _tc_body,
        out_shape=jax.ShapeDtypeStruct((B, OUT), f32),
    )(t2f.reshape(B, 1), m.reshape(GPAD, 1), w1, w2,
      w_q, b_q.reshape(1, H), w_k, b_k.reshape(1, H),
      w_v, b_v.reshape(1, H),
      w_l, b_l.reshape(1, H),
      w_lin, b_lin.reshape(1, OUT))


# speedup vs baseline: 1.0961x; 1.0196x over previous
import functools

import jax
import jax.numpy as jnp
from jax import lax
from jax.experimental import pallas as pl
from jax.experimental.pallas import tpu as pltpu
from jax.experimental.pallas import tpu_sc as plsc

N = 10000
E = 320000
NG = 200
EG = 600
G = 30
B = 64
H = 128
OUT = 16

NS = 16
SLICE = 640
NPAD = NS * SLICE
CH = 128
MCH = 512
TPE = E // NS
NCHUNK = TPE // MCH
TAIL = TPE - NCHUNK * MCH
NGPAD = 256
GCH = NGPAD // CH
GEC = 5
GPAD = 128


def _rsqrt16(x):
    i = lax.bitcast_convert_type(x, jnp.int32)
    i = jnp.int32(0x5F3759DF) - lax.shift_right_logical(i, 1)
    y = lax.bitcast_convert_type(i, jnp.float32)
    for _ in range(3):
        y = y * (1.5 - 0.5 * x * y * y)
    return y


def _sc_body(src_h, dst_h, gsrc_h, gdst_h, gseg_h, fetch_h,
             t2f_h, m_h,
             sidx_v, didx_v, xtab_v, vals_v, slice_v, invs_v, work_v,
             zer_v, ones_v,
             gsidx_v, gdidx_v, gseg_v, gx_v, ginvs_v, gt_v, gread_v,
             gvals_v, fidx_v, fout_v, mout_v, cnt_v, valsbuf_v, dumi_v,
             acc_sh, x_sh, gacc_sh, scnt_sh, ssum_sh, sem, sem2):
    tid = lax.axis_index("s")
    base = tid * SLICE
    ebase = tid * TPE

    def _drain_one(j, _):
        pltpu.make_async_copy(src_h.at[pl.ds(0, MCH)], dumi_v, sem).wait()
        return 0

    def _g_drain_one():
        pltpu.make_async_copy(src_h.at[pl.ds(0, CH)],
                              dumi_v.at[pl.ds(0, CH)], sem).wait()

    def _drain_tail():
        pltpu.make_async_copy(src_h.at[pl.ds(0, TAIL)],
                              dumi_v.at[pl.ds(0, TAIL)], sem).wait()

    for i in range(SLICE // 16):
        zer_v[pl.ds(16 * i, 16)] = jnp.zeros((16,), jnp.float32)
    for i in range(MCH // 16):
        ones_v[pl.ds(16 * i, 16)] = jnp.ones((16,), jnp.float32)

    is0 = tid == 0

    src_cp = pltpu.async_copy(src_h.at[pl.ds(ebase, TPE)], sidx_v, sem2)
    pltpu.sync_copy(dst_h.at[pl.ds(ebase, TPE)], didx_v)

    pltpu.sync_copy(zer_v, acc_sh.at[pl.ds(base, SLICE)])

    @pl.when(is0)
    def _g_stage():
        pltpu.sync_copy(gsrc_h, gsidx_v)
        pltpu.sync_copy(gdst_h, gdidx_v)
        pltpu.sync_copy(gseg_h, gseg_v)
        pltpu.sync_copy(fetch_h, fidx_v)
        pltpu.sync_copy(zer_v.at[pl.ds(0, NGPAD)], gacc_sh)
    plsc.subcore_barrier()

    def _g_read_transform(which):
        pltpu.sync_copy(gacc_sh, gread_v)
        for i in range(NGPAD // 16):
            u = gread_v[pl.ds(16 * i, 16)]
            if which == 0:
                y = _rsqrt16(jnp.maximum(u, 1.0))
                ginvs_v[pl.ds(16 * i, 16)] = y
                gx_v[pl.ds(16 * i, 16)] = u * y
            elif which == 1:
                y = ginvs_v[pl.ds(16 * i, 16)]
                gx_v[pl.ds(16 * i, 16)] = u * y * y
            else:
                y = ginvs_v[pl.ds(16 * i, 16)]
                gt_v[pl.ds(16 * i, 16)] = u * y
        if which != 2:
            pltpu.sync_copy(zer_v.at[pl.ds(0, NGPAD)], gacc_sh)

    def _g_drain():
        for _ in range(GEC):
            _g_drain_one()

    def _passA(j, _):
        pltpu.async_copy(ones_v,
                         acc_sh.at[didx_v.at[pl.ds(j * MCH, MCH)]],
                         sem, add=True)
        return 0
    lax.fori_loop(0, NCHUNK, _passA, 0)
    pltpu.async_copy(ones_v.at[pl.ds(0, TAIL)],
                     acc_sh.at[didx_v.at[pl.ds(NCHUNK * MCH, TAIL)]],
                     sem, add=True)

    @pl.when(is0)
    def _gA():
        for j in range(GEC):
            pltpu.async_copy(ones_v.at[pl.ds(0, CH)],
                             gacc_sh.at[gdidx_v.at[j]], sem, add=True)

    lax.fori_loop(0, NCHUNK, _drain_one, 0)
    _drain_tail()

    @pl.when(is0)
    def _gA2():
        _g_drain()
        _g_read_transform(0)

    src_cp.wait()
    plsc.subcore_barrier()

    pltpu.sync_copy(acc_sh.at[pl.ds(base, SLICE)], slice_v)
    for i in range(SLICE // 16):
        d = slice_v[pl.ds(16 * i, 16)]
        y = _rsqrt16(jnp.maximum(d, 1.0))
        invs_v[pl.ds(16 * i, 16)] = y
        work_v[pl.ds(16 * i, 16)] = d * y
    pltpu.sync_copy(zer_v, acc_sh.at[pl.ds(base, SLICE)])
    pltpu.sync_copy(work_v, x_sh.at[pl.ds(base, SLICE)])
    plsc.subcore_barrier()
    pltpu.sync_copy(x_sh, xtab_v)

    def _gather_scatter(j, _):
        off = j * MCH
        for i in range(MCH // 16):
            idx = sidx_v[pl.ds(off + 16 * i, 16)]
            valsbuf_v[pl.ds(off + 16 * i, 16)] = plsc.load_gather(xtab_v, [idx])
        pltpu.async_copy(valsbuf_v.at[pl.ds(off, MCH)],
                         acc_sh.at[didx_v.at[pl.ds(off, MCH)]],
                         sem, add=True)
        return 0

    def _gather_scatter_tail():
        off = NCHUNK * MCH
        for i in range(TAIL // 16):
            idx = sidx_v[pl.ds(off + 16 * i, 16)]
            valsbuf_v[pl.ds(off + 16 * i, 16)] = plsc.load_gather(xtab_v, [idx])
        pltpu.async_copy(valsbuf_v.at[pl.ds(off, TAIL)],
                         acc_sh.at[didx_v.at[pl.ds(off, TAIL)]],
                         sem, add=True)

    def _g_gather_scatter():
        for j in range(GEC):
            grow = gsidx_v.at[j]
            gvrow = gvals_v.at[j]
            for i in range(CH // 16):
                idx = grow[pl.ds(16 * i, 16)]
                gvrow[pl.ds(16 * i, 16)] = plsc.load_gather(gx_v, [idx])
            pltpu.async_copy(gvrow, gacc_sh.at[gdidx_v.at[j]], sem, add=True)

    lax.fori_loop(0, NCHUNK, _gather_scatter, 0)
    _gather_scatter_tail()

    @pl.when(is0)
    def _gB():
        _g_gather_scatter()

    lax.fori_loop(0, NCHUNK, _drain_one, 0)
    _drain_tail()

    @pl.when(is0)
    def _gB2():
        _g_drain()
        _g_read_transform(1)

    plsc.subcore_barrier()

    pltpu.sync_copy(acc_sh.at[pl.ds(base, SLICE)], slice_v)
    for i in range(SLICE // 16):
        u = slice_v[pl.ds(16 * i, 16)]
        y = invs_v[pl.ds(16 * i, 16)]
        work_v[pl.ds(16 * i, 16)] = u * y * y
    pltpu.sync_copy(zer_v, acc_sh.at[pl.ds(base, SLICE)])
    pltpu.sync_copy(work_v, x_sh.at[pl.ds(base, SLICE)])
    plsc.subcore_barrier()
    pltpu.sync_copy(x_sh, xtab_v)

    lax.fori_loop(0, NCHUNK, _gather_scatter, 0)
    _gather_scatter_tail()

    @pl.when(is0)
    def _gC():
        _g_gather_scatter()

    lax.fori_loop(0, NCHUNK, _drain_one, 0)
    _drain_tail()

    @pl.when(is0)
    def _gC2():
        _g_drain()
        _g_read_transform(2)

    plsc.subcore_barrier()

    pltpu.sync_copy(acc_sh.at[pl.ds(base, SLICE)], slice_v)
    for i in range(SLICE // 16):
        u = slice_v[pl.ds(16 * i, 16)]
        y = invs_v[pl.ds(16 * i, 16)]
        work_v[pl.ds(16 * i, 16)] = u * y
    pltpu.sync_copy(work_v, x_sh.at[pl.ds(base, SLICE)])
    plsc.subcore_barrier()

    @pl.when(tid == 0)
    def _tail():
        pltpu.sync_copy(x_sh, xtab_v)
        for i in range(B // 16):
            idx = fidx_v[pl.ds(16 * i, 16)]
            fout_v[pl.ds(16 * i, 16)] = plsc.load_gather(xtab_v, [idx])
        pltpu.sync_copy(fout_v, t2f_h)

        pltpu.sync_copy(zer_v.at[pl.ds(0, GPAD)], scnt_sh)
        pltpu.sync_copy(zer_v.at[pl.ds(0, GPAD)], ssum_sh)
        for j in range(GCH):
            pltpu.sync_copy(ones_v.at[pl.ds(0, CH)],
                        scnt_sh.at[gseg_v.at[j]], add=True)
            pltpu.sync_copy(gt_v.at[pl.ds(j * CH, CH)],
                            ssum_sh.at[gseg_v.at[j]], add=True)
        pltpu.sync_copy(scnt_sh, cnt_v)
        pltpu.sync_copy(ssum_sh, gread_v.at[pl.ds(0, GPAD)])
        for i in range(GPAD // 16):
            c = cnt_v[pl.ds(16 * i, 16)]
            s = gread_v[pl.ds(16 * i, 16)]
            mout_v[pl.ds(16 * i, 16)] = s / jnp.maximum(c, 1.0)
        pltpu.sync_copy(mout_v, m_h)


def _sc_call(*args):
    return functools.partial(
        pl.kernel,
        out_type=(jax.ShapeDtypeStruct((B,), jnp.float32),
                  jax.ShapeDtypeStruct((GPAD,), jnp.float32)),
        mesh=plsc.VectorSubcoreMesh(core_axis_name="c", subcore_axis_name="s",
                                    num_cores=1, num_subcores=NS),
        compiler_params=pltpu.CompilerParams(needs_layout_passes=False),
        scratch_types=[
        pltpu.VMEM((TPE,), jnp.int32),
        pltpu.VMEM((TPE,), jnp.int32),
        pltpu.VMEM((NPAD,), jnp.float32),
        pltpu.VMEM((CH,), jnp.float32),
        pltpu.VMEM((SLICE,), jnp.float32),
        pltpu.VMEM((SLICE,), jnp.float32),
        pltpu.VMEM((SLICE,), jnp.float32),
        pltpu.VMEM((SLICE,), jnp.float32),
        pltpu.VMEM((MCH,), jnp.float32),
        pltpu.VMEM((GEC, CH), jnp.int32),
        pltpu.VMEM((GEC, CH), jnp.int32),
        pltpu.VMEM((GCH, CH), jnp.int32),
        pltpu.VMEM((NGPAD,), jnp.float32),
        pltpu.VMEM((NGPAD,), jnp.float32),
        pltpu.VMEM((NGPAD,), jnp.float32),
        pltpu.VMEM((NGPAD,), jnp.float32),
        pltpu.VMEM((GEC, CH), jnp.float32),
        pltpu.VMEM((B,), jnp.int32),
        pltpu.VMEM((B,), jnp.float32),
        pltpu.VMEM((GPAD,), jnp.float32),
        pltpu.VMEM((GPAD,), jnp.float32),
        pltpu.VMEM((TPE,), jnp.float32),
        pltpu.VMEM((MCH,), jnp.int32),
        pltpu.VMEM_SHARED((NPAD,), jnp.float32),
        pltpu.VMEM_SHARED((NPAD,), jnp.float32),
        pltpu.VMEM_SHARED((NGPAD,), jnp.float32),
        pltpu.VMEM_SHARED((GPAD,), jnp.float32),
        pltpu.VMEM_SHARED((GPAD,), jnp.float32),
        pltpu.SemaphoreType.DMA,
        pltpu.SemaphoreType.DMA,
        ],
    )(_sc_body)(*args)


def _dotT(a, b):
    return lax.dot_general(a, b, (((1,), (1,)), ((), ())),
                           preferred_element_type=jnp.float32,
                           precision=lax.Precision.HIGHEST)


def _tc_body(t2f_ref, m_ref, w1_ref, w2_ref, wq_ref, bq_ref, wk_ref,
             bk_ref, wv_ref, bv_ref, wl_ref, bl_ref,
             wlin_ref, blin_ref, out_ref):
    w1p = jnp.maximum(w1_ref[...], 0.0)
    rv = jnp.dot(w1p, w2_ref[...], preferred_element_type=jnp.float32,
                 precision=lax.Precision.HIGHEST)
    rv = jnp.maximum(rv, 0.0)
    hf = t2f_ref[...] * rv
    hg = m_ref[0:32, :] * rv
    q = _dotT(hf, wq_ref[...]) + bq_ref[...]
    k = _dotT(hg, wk_ref[...]) + bk_ref[...]
    v = _dotT(hg, wv_ref[...]) + bv_ref[...]
    s = _dotT(q, k)
    col = lax.broadcasted_iota(jnp.int32, (B, 32), 1)
    s = jnp.where(col < G, s, -1e30)
    s = s - jnp.max(s, axis=1, keepdims=True)
    p = jnp.exp(s)
    p = p / jnp.sum(p, axis=1, keepdims=True)
    ctx = jnp.dot(p, v, preferred_element_type=jnp.float32,
                  precision=lax.Precision.HIGHEST)
    wl = wl_ref[...]
    ho = _dotT(ctx, wl[:, :H]) + _dotT(hf, wl[:, H:]) + bl_ref[...]
    out_ref[...] = _dotT(ho, wlin_ref[...]) + blin_ref[...]


def kernel(edge_index, fetch_idx, g_edge_index, g_seg_ids, w1, b1, w2, b2,
           w_q, b_q, w_k, b_k, w_v, b_v, w_l, b_l, w_lin, b_lin):
    f32 = jnp.float32
    i32 = jnp.int32
    srcp = edge_index[0].astype(i32)
    dstp = edge_index[1].astype(i32)
    gpad = GEC * CH - EG
    gsrcp = jnp.concatenate([g_edge_index[0].astype(i32),
                             jnp.zeros((gpad,), i32)]).reshape(GEC, CH)
    gdstp = jnp.concatenate([g_edge_index[1].astype(i32),
                             jnp.full((gpad,), NGPAD - 1, i32)]).reshape(GEC, CH)
    spad = GCH * CH - NG
    gsegp = jnp.concatenate([g_seg_ids.astype(i32),
                             jnp.full((spad,), GPAD - 1, i32)]).reshape(GCH, CH)

    t2f, m = _sc_call(srcp, dstp, gsrcp, gdstp, gsegp, fetch_idx.astype(i32))

    return pl.pallas_call(
        _tc_body,
        out_shape=jax.ShapeDtypeStruct((B, OUT), f32),
    )(t2f.reshape(B, 1), m.reshape(GPAD, 1), w1, w2,
      w_q, b_q.reshape(1, H), w_k, b_k.reshape(1, H),
      w_v, b_v.reshape(1, H),
      w_l, b_l.reshape(1, H),
      w_lin, b_lin.reshape(1, OUT))
